# Initial kernel scaffold; baseline (speedup 1.0000x reference)
import jax
import jax.numpy as jnp
from jax.experimental import pallas as pl


def kernel(x, edge_index, W1, b1, g1, be1, W2, b2, g2, be2, W3, b3):
    raise NotImplementedError("write your pallas kernel here")


# speedup vs baseline: 14.8481x; 14.8481x over previous
import functools

import jax
import jax.numpy as jnp
from jax import lax
from jax.experimental import pallas as pl
from jax.experimental.pallas import tpu as pltpu
from jax.experimental.pallas import tpu_sc as plsc

NC = 2
NS = 16
DW = 16
ZR = 125


@functools.lru_cache(maxsize=None)
def _deg_pass(n, e, ch):
    ept = e // (NC * NS)
    nchunks = ept // ch
    rpt = n // NS
    mesh = plsc.VectorSubcoreMesh(core_axis_name="c", subcore_axis_name="s",
                                  num_cores=NC, num_subcores=NS)

    def body(dst_hbm, out_hbm, acc, ones_v, idx_v, zb):
        c = lax.axis_index("c")
        s = lax.axis_index("s")
        w = s * NC + c

        def init_z(r, _):
            zb[r, :] = jnp.zeros((DW,), jnp.float32)
            return 0

        lax.fori_loop(0, ZR, init_z, 0)

        def init_o(r, _):
            ones_v[r, :] = jnp.ones((DW,), jnp.float32)
            return 0

        lax.fori_loop(0, ch, init_o, 0)

        def zrow(k, _):
            pltpu.sync_copy(zb, acc.at[pl.ds(s * rpt + k * ZR, ZR)])
            return 0

        lax.fori_loop(0, rpt // ZR, zrow, 0)
        plsc.subcore_barrier()

        def step(i, _):
            off = w * ept + i * ch
            pltpu.sync_copy(dst_hbm.at[pl.ds(off, ch)], idx_v)
            pltpu.sync_copy(ones_v, acc.at[idx_v], add=True)
            return 0

        lax.fori_loop(0, nchunks, step, 0)
        plsc.subcore_barrier()
        pltpu.sync_copy(acc.at[pl.ds(s * rpt, rpt)], out_hbm.at[c, s])

    return pl.kernel(
        body,
        out_type=jax.ShapeDtypeStruct((NC, NS, n // NS, DW), jnp.float32),
        mesh=mesh,
        compiler_params=pltpu.CompilerParams(use_tc_tiling_on_sc=False),
        scratch_types=[
            pltpu.VMEM_SHARED((n, DW), jnp.float32),
            pltpu.VMEM((ch, DW), jnp.float32),
            pltpu.VMEM((ch,), jnp.int32),
            pltpu.VMEM((ZR, DW), jnp.float32),
        ],
    )


@functools.lru_cache(maxsize=None)
def _conv_pass(n, e, fh, ch):
    ept = e // NS
    nchunks = ept // ch
    nvec = ch // 16
    rpt = n // NS
    mesh = plsc.VectorSubcoreMesh(core_axis_name="c", subcore_axis_name="s",
                                  num_cores=NC, num_subcores=NS)

    def body(u_hbm, src_hbm, dst_hbm, out_hbm, acc, rows_v, src_v, dst_v, zb, sem):
        c = lax.axis_index("c")
        s = lax.axis_index("s")
        cn = c * n

        def init_z(r, _):
            for j in range(fh // 16):
                zb[r, pl.ds(j * 16, 16)] = jnp.zeros((16,), jnp.float32)
            return 0

        lax.fori_loop(0, ZR, init_z, 0)

        def zrow(k, _):
            pltpu.sync_copy(zb, acc.at[pl.ds(s * rpt + k * ZR, ZR)])
            return 0

        lax.fori_loop(0, rpt // ZR, zrow, 0)
        plsc.subcore_barrier()

        def step(i, _):
            off = s * ept + i * ch
            pltpu.sync_copy(src_hbm.at[pl.ds(off, ch)], src_v)
            pltpu.sync_copy(dst_hbm.at[pl.ds(off, ch)], dst_v)

            def addoff(j, _):
                src_v[pl.ds(j * 16, 16)] = src_v[pl.ds(j * 16, 16)] + cn
                return 0

            lax.fori_loop(0, nvec, addoff, 0)
            pltpu.async_copy(u_hbm.at[src_v], rows_v, sem).wait()
            pltpu.sync_copy(rows_v, acc.at[dst_v], add=True)
            return 0

        lax.fori_loop(0, nchunks, step, 0)
        plsc.subcore_barrier()
        pltpu.sync_copy(acc.at[pl.ds(s * rpt, rpt)], out_hbm.at[c, s])

    return pl.kernel(
        body,
        out_type=jax.ShapeDtypeStruct((NC, NS, n // NS, fh), jnp.float32),
        mesh=mesh,
        compiler_params=pltpu.CompilerParams(use_tc_tiling_on_sc=False),
        scratch_types=[
            pltpu.VMEM_SHARED((n, fh), jnp.float32),
            pltpu.VMEM((ch, fh), jnp.float32),
            pltpu.VMEM((ch,), jnp.int32),
            pltpu.VMEM((ch,), jnp.int32),
            pltpu.VMEM((ZR, fh), jnp.float32),
            pltpu.SemaphoreType.DMA,
        ],
    )


_R = 1000


def _dinv(d0_ref, d1_ref):
    deg = d0_ref[...] + d1_ref[...] + 1.0
    return lax.rsqrt(jnp.maximum(deg, 1.0))


def _scale_split(v, d0, d1):
    n, f = v.shape
    fh = f // 2
    nb = n // _R

    def body(v_ref, d0_ref, d1_ref, o_ref):
        u = v_ref[...] * _dinv(d0_ref, d1_ref)
        o_ref[0, :, :] = u[:, :fh]
        o_ref[1, :, :] = u[:, fh:]

    out = pl.pallas_call(
        body,
        grid=(nb,),
        in_specs=[
            pl.BlockSpec((_R, f), lambda j: (j, 0)),
            pl.BlockSpec((_R, 1), lambda j: (j, 0)),
            pl.BlockSpec((_R, 1), lambda j: (j, 0)),
        ],
        out_specs=pl.BlockSpec((NC, _R, fh), lambda j: (0, j, 0)),
        out_shape=jax.ShapeDtypeStruct((NC, n, fh), jnp.float32),
    )(v, d0, d1)
    return out.reshape(NC * n, fh)


def _stage_b(a0, a1, x, d0, d1, W1, b1, g1, be1):
    n, fin = x.shape
    h = W1.shape[1]
    hh = h // 2
    nb = n // _R

    def body(a0_ref, a1_ref, x_ref, d0_ref, d1_ref, w_ref, b_ref, g_ref,
             be_ref, h1_ref):
        dinv = _dinv(d0_ref, d1_ref)
        agg = jnp.concatenate([a0_ref[...], a1_ref[...]], axis=1)
        pre = dinv * (agg + dinv * x_ref[...])
        z = jnp.dot(pre, w_ref[...], preferred_element_type=jnp.float32,
                    precision=lax.Precision.HIGHEST) + b_ref[...]
        h1_ref[...] = jnp.maximum(g_ref[...] * z + be_ref[...], 0.0)

    return pl.pallas_call(
        body,
        grid=(nb, NC),
        in_specs=[
            pl.BlockSpec((_R, fin // 2), lambda j, h_: (j, 0)),
            pl.BlockSpec((_R, fin // 2), lambda j, h_: (j, 0)),
            pl.BlockSpec((_R, fin), lambda j, h_: (j, 0)),
            pl.BlockSpec((_R, 1), lambda j, h_: (j, 0)),
            pl.BlockSpec((_R, 1), lambda j, h_: (j, 0)),
            pl.BlockSpec((fin, hh), lambda j, h_: (0, h_)),
            pl.BlockSpec((1, hh), lambda j, h_: (0, h_)),
            pl.BlockSpec((1, hh), lambda j, h_: (0, h_)),
            pl.BlockSpec((1, hh), lambda j, h_: (0, h_)),
        ],
        out_specs=pl.BlockSpec((_R, hh), lambda j, h_: (j, h_)),
        out_shape=jax.ShapeDtypeStruct((n, h), jnp.float32),
    )(a0, a1, x, d0, d1, W1, b1, g1, be1)


def _stage_c(a0, a1, a2, a3, h1, d0, d1, W2, b2, g2, be2, W3p):
    n, h = h1.shape
    hh = h // 2
    op = W3p.shape[1]
    nb = n // _R

    def body(a0_ref, a1_ref, a2_ref, a3_ref, h1_ref, d0_ref, d1_ref,
             w2_ref, b_ref, g_ref, be_ref, w3_ref, h2_ref, p_ref):
        h_ = pl.program_id(1)
        dinv = _dinv(d0_ref, d1_ref)
        agg = jnp.concatenate(
            [a0_ref[...], a1_ref[...], a2_ref[...], a3_ref[...]], axis=1)
        pre = dinv * (agg + dinv * h1_ref[...])
        z = jnp.dot(pre, w2_ref[...], preferred_element_type=jnp.float32,
                    precision=lax.Precision.HIGHEST) + b_ref[...]
        h2 = jnp.maximum(g_ref[...] * z + be_ref[...], 0.0)
        h2_ref[...] = h2
        contrib = jnp.dot(h2, w3_ref[...], preferred_element_type=jnp.float32,
                          precision=lax.Precision.HIGHEST)

        @pl.when(h_ == 0)
        def _():
            p_ref[...] = jnp.zeros_like(p_ref)

        p_ref[...] += contrib

    return pl.pallas_call(
        body,
        grid=(nb, NC),
        in_specs=[
            pl.BlockSpec((_R, hh // 2), lambda j, h_: (j, 0)),
            pl.BlockSpec((_R, hh // 2), lambda j, h_: (j, 0)),
            pl.BlockSpec((_R, hh // 2), lambda j, h_: (j, 0)),
            pl.BlockSpec((_R, hh // 2), lambda j, h_: (j, 0)),
            pl.BlockSpec((_R, h), lambda j, h_: (j, 0)),
            pl.BlockSpec((_R, 1), lambda j, h_: (j, 0)),
            pl.BlockSpec((_R, 1), lambda j, h_: (j, 0)),
            pl.BlockSpec((h, hh), lambda j, h_: (0, h_)),
            pl.BlockSpec((1, hh), lambda j, h_: (0, h_)),
            pl.BlockSpec((1, hh), lambda j, h_: (0, h_)),
            pl.BlockSpec((1, hh), lambda j, h_: (0, h_)),
            pl.BlockSpec((hh, op), lambda j, h_: (h_, 0)),
        ],
        out_specs=[
            pl.BlockSpec((_R, hh), lambda j, h_: (j, h_)),
            pl.BlockSpec((_R, op), lambda j, h_: (j, 0)),
        ],
        out_shape=[
            jax.ShapeDtypeStruct((n, h), jnp.float32),
            jax.ShapeDtypeStruct((n, op), jnp.float32),
        ],
    )(a0, a1, a2, a3, h1, d0, d1, W2, b2, g2, be2, W3p)


def _stage_d(a0, a1, p, d0, d1, b3, out_dim):
    n, op = p.shape
    nb = n // _R

    def body(a0_ref, a1_ref, p_ref, d0_ref, d1_ref, b_ref, o_ref):
        dinv = _dinv(d0_ref, d1_ref)
        agg = jnp.concatenate([a0_ref[...], a1_ref[...]], axis=1)
        z = dinv * (agg + dinv * p_ref[...])
        logits = z[:, :out_dim] + b_ref[...]
        m = jnp.max(logits, axis=1, keepdims=True)
        ex = jnp.exp(logits - m)
        lse = jnp.log(jnp.sum(ex, axis=1, keepdims=True)) + m
        o_ref[...] = logits - lse

    return pl.pallas_call(
        body,
        grid=(nb,),
        in_specs=[
            pl.BlockSpec((_R, op // 2), lambda j: (j, 0)),
            pl.BlockSpec((_R, op // 2), lambda j: (j, 0)),
            pl.BlockSpec((_R, op), lambda j: (j, 0)),
            pl.BlockSpec((_R, 1), lambda j: (j, 0)),
            pl.BlockSpec((_R, 1), lambda j: (j, 0)),
            pl.BlockSpec((1, out_dim), lambda j: (0, 0)),
        ],
        out_specs=pl.BlockSpec((_R, out_dim), lambda j: (j, 0)),
        out_shape=jax.ShapeDtypeStruct((n, out_dim), jnp.float32),
    )(a0, a1, p, d0, d1, b3)


def kernel(x, edge_index, W1, b1, g1, be1, W2, b2, g2, be2, W3, b3):
    n, fin = x.shape
    e = edge_index.shape[1]
    h = W1.shape[1]
    out_dim = W3.shape[1]
    op = 64

    src = edge_index[0]
    dst = edge_index[1]
    deg2 = _deg_pass(n, e, 1000)(dst).reshape(NC, n, DW)
    d0 = deg2[0, :, :1]
    d1 = deg2[1, :, :1]

    u0 = _scale_split(x, d0, d1)
    a0 = _conv_pass(n, e, fin // 2, 800)(u0, src, dst).reshape(NC, n, fin // 2)
    h1 = _stage_b(a0[0], a0[1], x, d0, d1, W1,
                  b1.reshape(1, -1), g1.reshape(1, -1), be1.reshape(1, -1))
    hh = h // 2
    u1a = _scale_split(h1[:, :hh], d0, d1)
    u1b = _scale_split(h1[:, hh:], d0, d1)
    a1a = _conv_pass(n, e, hh // 2, 800)(u1a, src, dst).reshape(NC, n, hh // 2)
    a1b = _conv_pass(n, e, hh // 2, 800)(u1b, src, dst).reshape(NC, n, hh // 2)
    W3p = jnp.pad(W3, ((0, 0), (0, op - out_dim)))
    h2, p = _stage_c(a1a[0], a1a[1], a1b[0], a1b[1], h1, d0, d1, W2,
                     b2.reshape(1, -1), g2.reshape(1, -1), be2.reshape(1, -1),
                     W3p)
    u2 = _scale_split(p, d0, d1)
    a2 = _conv_pass(n, e, op // 2, 800)(u2, src, dst).reshape(NC, n, op // 2)
    out = _stage_d(a2[0], a2[1], p, d0, d1, b3.reshape(1, -1), out_dim)
    return (out, h2)


# speedup vs baseline: 19.3178x; 1.3010x over previous
import functools

import jax
import jax.numpy as jnp
from jax import lax
from jax.experimental import pallas as pl
from jax.experimental.pallas import tpu as pltpu
from jax.experimental.pallas import tpu_sc as plsc

NC = 2
NS = 16
DW = 16
ZR = 125


@functools.lru_cache(maxsize=None)
def _deg_pass(n, e, ch):
    ept = e // (NC * NS)
    nchunks = ept // ch
    rpt = n // NS
    mesh = plsc.VectorSubcoreMesh(core_axis_name="c", subcore_axis_name="s",
                                  num_cores=NC, num_subcores=NS)

    def body(dst_hbm, out_hbm, acc, ones_v, idx_v, zb):
        c = lax.axis_index("c")
        s = lax.axis_index("s")
        w = s * NC + c

        def init_z(r, _):
            zb[r, :] = jnp.zeros((DW,), jnp.float32)
            return 0

        lax.fori_loop(0, ZR, init_z, 0)

        def init_o(r, _):
            ones_v[r, :] = jnp.ones((DW,), jnp.float32)
            return 0

        lax.fori_loop(0, ch, init_o, 0)

        def zrow(k, _):
            pltpu.sync_copy(zb, acc.at[pl.ds(s * rpt + k * ZR, ZR)])
            return 0

        lax.fori_loop(0, rpt // ZR, zrow, 0)
        plsc.subcore_barrier()

        def step(i, _):
            off = w * ept + i * ch
            pltpu.sync_copy(dst_hbm.at[pl.ds(off, ch)], idx_v)
            pltpu.sync_copy(ones_v, acc.at[idx_v], add=True)
            return 0

        lax.fori_loop(0, nchunks, step, 0)
        plsc.subcore_barrier()
        pltpu.sync_copy(acc.at[pl.ds(s * rpt, rpt)], out_hbm.at[c, s])

    return pl.kernel(
        body,
        out_type=jax.ShapeDtypeStruct((NC, NS, n // NS, DW), jnp.float32),
        mesh=mesh,
        compiler_params=pltpu.CompilerParams(use_tc_tiling_on_sc=False),
        scratch_types=[
            pltpu.VMEM_SHARED((n, DW), jnp.float32),
            pltpu.VMEM((ch, DW), jnp.float32),
            pltpu.VMEM((ch,), jnp.int32),
            pltpu.VMEM((ZR, DW), jnp.float32),
        ],
    )


@functools.lru_cache(maxsize=None)
def _conv_pass(n, e, fh, ch):
    ept = e // NS
    nchunks = ept // ch
    nvec = ch // 16
    rpt = n // NS
    mesh = plsc.VectorSubcoreMesh(core_axis_name="c", subcore_axis_name="s",
                                  num_cores=NC, num_subcores=NS)

    def body(u_hbm, src_hbm, dst_hbm, out_hbm, acc,
             rows0, rows1, src0, src1, dst0, dst1, zb, sem0, sem1):
        c = lax.axis_index("c")
        s = lax.axis_index("s")
        cn = c * n

        def init_z(r, _):
            for j in range(fh // 16):
                zb[r, pl.ds(j * 16, 16)] = jnp.zeros((16,), jnp.float32)
            return 0

        lax.fori_loop(0, ZR, init_z, 0)

        def zrow(k, _):
            pltpu.sync_copy(zb, acc.at[pl.ds(s * rpt + k * ZR, ZR)])
            return 0

        lax.fori_loop(0, rpt // ZR, zrow, 0)
        plsc.subcore_barrier()

        def fire(i, rows_v, src_v, dst_v, sem):
            off = s * ept + i * ch
            pltpu.sync_copy(src_hbm.at[pl.ds(off, ch)], src_v)
            pltpu.sync_copy(dst_hbm.at[pl.ds(off, ch)], dst_v)

            def addoff(j, _):
                src_v[pl.ds(j * 16, 16)] = src_v[pl.ds(j * 16, 16)] + cn
                return 0

            lax.fori_loop(0, nvec, addoff, 0)
            pltpu.async_copy(u_hbm.at[src_v], rows_v, sem)

        def finish(i, rows_v, src_v, dst_v, sem):
            pltpu.make_async_copy(u_hbm.at[src_v], rows_v, sem).wait()
            pltpu.sync_copy(rows_v, acc.at[dst_v], add=True)

        fire(0, rows0, src0, dst0, sem0)

        def step(i, _):
            @pl.when(lax.rem(i, 2) == 0)
            def _():
                @pl.when(i + 1 < nchunks)
                def _():
                    fire(i + 1, rows1, src1, dst1, sem1)
                finish(i, rows0, src0, dst0, sem0)

            @pl.when(lax.rem(i, 2) == 1)
            def _():
                @pl.when(i + 1 < nchunks)
                def _():
                    fire(i + 1, rows0, src0, dst0, sem0)
                finish(i, rows1, src1, dst1, sem1)

            return 0

        lax.fori_loop(0, nchunks, step, 0)
        plsc.subcore_barrier()
        pltpu.sync_copy(acc.at[pl.ds(s * rpt, rpt)], out_hbm.at[c, s])

    return pl.kernel(
        body,
        out_type=jax.ShapeDtypeStruct((NC, NS, n // NS, fh), jnp.float32),
        mesh=mesh,
        compiler_params=pltpu.CompilerParams(use_tc_tiling_on_sc=False),
        scratch_types=[
            pltpu.VMEM_SHARED((n, fh), jnp.float32),
            pltpu.VMEM((ch, fh), jnp.float32),
            pltpu.VMEM((ch, fh), jnp.float32),
            pltpu.VMEM((ch,), jnp.int32),
            pltpu.VMEM((ch,), jnp.int32),
            pltpu.VMEM((ch,), jnp.int32),
            pltpu.VMEM((ch,), jnp.int32),
            pltpu.VMEM((ZR, fh), jnp.float32),
            pltpu.SemaphoreType.DMA,
            pltpu.SemaphoreType.DMA,
        ],
    )


_R = 1000


def _dinv(d0_ref, d1_ref):
    deg = d0_ref[...] + d1_ref[...] + 1.0
    return lax.rsqrt(jnp.maximum(deg, 1.0))


def _scale_split(v, d0, d1):
    n, f = v.shape
    fh = f // 2
    nb = n // _R

    def body(v_ref, d0_ref, d1_ref, o_ref):
        u = v_ref[...] * _dinv(d0_ref, d1_ref)
        o_ref[0, :, :] = u[:, :fh]
        o_ref[1, :, :] = u[:, fh:]

    out = pl.pallas_call(
        body,
        grid=(nb,),
        in_specs=[
            pl.BlockSpec((_R, f), lambda j: (j, 0)),
            pl.BlockSpec((_R, 1), lambda j: (j, 0)),
            pl.BlockSpec((_R, 1), lambda j: (j, 0)),
        ],
        out_specs=pl.BlockSpec((NC, _R, fh), lambda j: (0, j, 0)),
        out_shape=jax.ShapeDtypeStruct((NC, n, fh), jnp.float32),
    )(v, d0, d1)
    return out.reshape(NC * n, fh)


def _stage_b(a0, a1, x, d0, d1, W1, b1, g1, be1):
    n, fin = x.shape
    h = W1.shape[1]
    hh = h // 2
    nb = n // _R

    fq = h // 4

    def body(a0_ref, a1_ref, x_ref, d0_ref, d1_ref, w_ref, b_ref, g_ref,
             be_ref, h1_ref, u1_ref):
        dinv = _dinv(d0_ref, d1_ref)
        agg = jnp.concatenate([a0_ref[...], a1_ref[...]], axis=1)
        pre = dinv * (agg + dinv * x_ref[...])
        z = jnp.dot(pre, w_ref[...], preferred_element_type=jnp.float32,
                    precision=lax.Precision.HIGHEST) + b_ref[...]
        h1 = jnp.maximum(g_ref[...] * z + be_ref[...], 0.0)
        h1_ref[...] = h1
        u1 = dinv * h1
        for q in range(4):
            u1_ref[q, :, :] = u1[:, q * fq:(q + 1) * fq]

    return pl.pallas_call(
        body,
        grid=(nb,),
        in_specs=[
            pl.BlockSpec((_R, fin // 2), lambda j: (j, 0)),
            pl.BlockSpec((_R, fin // 2), lambda j: (j, 0)),
            pl.BlockSpec((_R, fin), lambda j: (j, 0)),
            pl.BlockSpec((_R, 1), lambda j: (j, 0)),
            pl.BlockSpec((_R, 1), lambda j: (j, 0)),
            pl.BlockSpec((fin, h), lambda j: (0, 0)),
            pl.BlockSpec((1, h), lambda j: (0, 0)),
            pl.BlockSpec((1, h), lambda j: (0, 0)),
            pl.BlockSpec((1, h), lambda j: (0, 0)),
        ],
        out_specs=[
            pl.BlockSpec((_R, h), lambda j: (j, 0)),
            pl.BlockSpec((4, _R, fq), lambda j: (0, j, 0)),
        ],
        out_shape=[
            jax.ShapeDtypeStruct((n, h), jnp.float32),
            jax.ShapeDtypeStruct((4, n, fq), jnp.float32),
        ],
    )(a0, a1, x, d0, d1, W1, b1, g1, be1)


def _stage_c(a0, a1, a2, a3, h1, d0, d1, W2, b2, g2, be2, W3p):
    n, h = h1.shape
    hh = h // 2
    op = W3p.shape[1]
    nb = n // _R

    oh = op // 2

    def body(a0_ref, a1_ref, a2_ref, a3_ref, h1_ref, d0_ref, d1_ref,
             w2_ref, b_ref, g_ref, be_ref, w3_ref, h2_ref, p_ref, u2_ref):
        dinv = _dinv(d0_ref, d1_ref)
        agg = jnp.concatenate(
            [a0_ref[...], a1_ref[...], a2_ref[...], a3_ref[...]], axis=1)
        pre = dinv * (agg + dinv * h1_ref[...])
        z = jnp.dot(pre, w2_ref[...], preferred_element_type=jnp.float32,
                    precision=lax.Precision.HIGHEST) + b_ref[...]
        h2 = jnp.maximum(g_ref[...] * z + be_ref[...], 0.0)
        h2_ref[...] = h2
        p = jnp.dot(h2, w3_ref[...], preferred_element_type=jnp.float32,
                    precision=lax.Precision.HIGHEST)
        p_ref[...] = p
        u2 = dinv * p
        u2_ref[0, :, :] = u2[:, :oh]
        u2_ref[1, :, :] = u2[:, oh:]

    return pl.pallas_call(
        body,
        grid=(nb,),
        in_specs=[
            pl.BlockSpec((_R, hh // 2), lambda j: (j, 0)),
            pl.BlockSpec((_R, hh // 2), lambda j: (j, 0)),
            pl.BlockSpec((_R, hh // 2), lambda j: (j, 0)),
            pl.BlockSpec((_R, hh // 2), lambda j: (j, 0)),
            pl.BlockSpec((_R, h), lambda j: (j, 0)),
            pl.BlockSpec((_R, 1), lambda j: (j, 0)),
            pl.BlockSpec((_R, 1), lambda j: (j, 0)),
            pl.BlockSpec((h, h), lambda j: (0, 0)),
            pl.BlockSpec((1, h), lambda j: (0, 0)),
            pl.BlockSpec((1, h), lambda j: (0, 0)),
            pl.BlockSpec((1, h), lambda j: (0, 0)),
            pl.BlockSpec((h, op), lambda j: (0, 0)),
        ],
        out_specs=[
            pl.BlockSpec((_R, h), lambda j: (j, 0)),
            pl.BlockSpec((_R, op), lambda j: (j, 0)),
            pl.BlockSpec((NC, _R, oh), lambda j: (0, j, 0)),
        ],
        out_shape=[
            jax.ShapeDtypeStruct((n, h), jnp.float32),
            jax.ShapeDtypeStruct((n, op), jnp.float32),
            jax.ShapeDtypeStruct((NC, n, oh), jnp.float32),
        ],
    )(a0, a1, a2, a3, h1, d0, d1, W2, b2, g2, be2, W3p)


def _stage_d(a0, a1, p, d0, d1, b3, out_dim):
    n, op = p.shape
    nb = n // _R

    def body(a0_ref, a1_ref, p_ref, d0_ref, d1_ref, b_ref, o_ref):
        dinv = _dinv(d0_ref, d1_ref)
        agg = jnp.concatenate([a0_ref[...], a1_ref[...]], axis=1)
        z = dinv * (agg + dinv * p_ref[...])
        logits = z[:, :out_dim] + b_ref[...]
        m = jnp.max(logits, axis=1, keepdims=True)
        ex = jnp.exp(logits - m)
        lse = jnp.log(jnp.sum(ex, axis=1, keepdims=True)) + m
        o_ref[...] = logits - lse

    return pl.pallas_call(
        body,
        grid=(nb,),
        in_specs=[
            pl.BlockSpec((_R, op // 2), lambda j: (j, 0)),
            pl.BlockSpec((_R, op // 2), lambda j: (j, 0)),
            pl.BlockSpec((_R, op), lambda j: (j, 0)),
            pl.BlockSpec((_R, 1), lambda j: (j, 0)),
            pl.BlockSpec((_R, 1), lambda j: (j, 0)),
            pl.BlockSpec((1, out_dim), lambda j: (0, 0)),
        ],
        out_specs=pl.BlockSpec((_R, out_dim), lambda j: (j, 0)),
        out_shape=jax.ShapeDtypeStruct((n, out_dim), jnp.float32),
    )(a0, a1, p, d0, d1, b3)


def kernel(x, edge_index, W1, b1, g1, be1, W2, b2, g2, be2, W3, b3):
    n, fin = x.shape
    e = edge_index.shape[1]
    h = W1.shape[1]
    out_dim = W3.shape[1]
    op = 64

    src = edge_index[0]
    dst = edge_index[1]
    deg2 = _deg_pass(n, e, 1000)(dst).reshape(NC, n, DW)
    d0 = deg2[0, :, :1]
    d1 = deg2[1, :, :1]

    u0 = _scale_split(x, d0, d1)
    a0 = _conv_pass(n, e, fin // 2, 400)(u0, src, dst).reshape(NC, n, fin // 2)
    h1, u1q = _stage_b(a0[0], a0[1], x, d0, d1, W1,
                       b1.reshape(1, -1), g1.reshape(1, -1),
                       be1.reshape(1, -1))
    hh = h // 2
    fq = h // 4
    u1a = u1q[:2].reshape(NC * n, fq)
    u1b = u1q[2:].reshape(NC * n, fq)
    a1a = _conv_pass(n, e, fq, 400)(u1a, src, dst).reshape(NC, n, fq)
    a1b = _conv_pass(n, e, fq, 400)(u1b, src, dst).reshape(NC, n, fq)
    W3p = jnp.pad(W3, ((0, 0), (0, op - out_dim)))
    h2, p, u2q = _stage_c(a1a[0], a1a[1], a1b[0], a1b[1], h1, d0, d1, W2,
                          b2.reshape(1, -1), g2.reshape(1, -1),
                          be2.reshape(1, -1), W3p)
    u2 = u2q.reshape(NC * n, op // 2)
    a2 = _conv_pass(n, e, op // 2, 800)(u2, src, dst).reshape(NC, n, op // 2)
    out = _stage_d(a2[0], a2[1], p, d0, d1, b3.reshape(1, -1), out_dim)
    return (out, h2)
